# baseline (speedup 1.0000x reference)
import functools

import jax
import jax.numpy as jnp
from jax import lax
from jax.experimental import pallas as pl
from jax.experimental.pallas import tpu as pltpu
from jax.experimental.pallas import tpu_sc as plsc

_K = 100
_B = 128
_N = 100000

_NC, _NS, _L = 2, 16, 16
_NW = _NC * _NS
_ROWS_PER_TILE = _B // _NW
_FBINS = 1 << 14
_CBINS = _FBINS // _L
_UNROLL = 16
_NVEC = _N // _L

_TCCHUNK = 10000
_TCGRID = _N // _TCCHUNK


def _keys_of(v):
    s = lax.bitcast_convert_type(v, jnp.int32)
    return s ^ (lax.shift_right_arithmetic(s, 31) & jnp.int32(0x7FFFFFFF))


def _suffix_find(v, acc, k):
    suffix = lax.rev(jnp.cumsum(lax.rev(v, (0,))), (0,))
    ok = (acc + suffix) >= k
    c = jnp.sum(ok.astype(jnp.int32))
    above = acc + jnp.sum(jnp.where(ok, 0, v))
    return c - 1, above


def _walk(hist, start_chunk, k):
    def chunk_sum(j):
        off = pl.multiple_of(j * _L, _L)
        return jnp.sum(hist[pl.ds(off, _L)])

    def cond(st):
        j, acc, s = st
        return acc + s < k

    def body(st):
        j, acc, s = st
        return j - 1, acc + s, chunk_sum(j - 1)

    j, acc, _ = lax.while_loop(
        cond, body, (start_chunk, jnp.int32(0), chunk_sum(start_chunk))
    )
    v = hist[pl.ds(pl.multiple_of(j * _L, _L), _L)]
    ci, above = _suffix_find(v, acc, k)
    return j * _L + ci, above


def _sc_body(nbf_hbm, out_hbm, row_buf, fine, coarse, res_buf):
    wid = lax.axis_index("s") * _NC + lax.axis_index("c")
    lanes = lax.iota(jnp.int32, _L)
    ones = jnp.ones((_L,), jnp.int32)
    zeros16 = jnp.zeros((_L,), jnp.int32)

    def row_body(j, res_keys):
        row = wid * _ROWS_PER_TILE + j
        with jax.named_scope("dma_row"):
            pltpu.sync_copy(nbf_hbm.at[row], row_buf)

        with jax.named_scope("clear"):
            @plsc.parallel_loop(0, _FBINS // _L, unroll=8)
            def _clear_f(i):
                fine[pl.ds(pl.multiple_of(i * _L, _L), _L)] = zeros16

        with jax.named_scope("pass1"):
            @plsc.parallel_loop(
                0, _NVEC, unroll=_UNROLL, carry=jnp.zeros((_L,), jnp.int32)
            )
            def mx(i, mx):
                off = pl.multiple_of(i * _L, _L)
                key = _keys_of(row_buf[pl.ds(off, _L)])
                row_buf[pl.ds(off, _L)] = lax.bitcast_convert_type(
                    key, jnp.float32
                )
                idx = lax.shift_right_arithmetic(key, 18) + jnp.int32(8192)
                plsc.addupdate_scatter(fine, [idx], ones)
                return jnp.maximum(mx, idx)

        with jax.named_scope("scan1"):
            start = lax.shift_right_logical(jnp.max(mx), 4)
            b1, above1 = _walk(fine, start, jnp.int32(_K))
            k2 = jnp.int32(_K) - above1
            b1s = b1 - jnp.int32(8192)

        with jax.named_scope("clear2"):
            @plsc.parallel_loop(0, _FBINS // _L, unroll=8)
            def _clear_f2(i):
                fine[pl.ds(pl.multiple_of(i * _L, _L), _L)] = zeros16

            @plsc.parallel_loop(0, _CBINS // _L, unroll=8)
            def _clear_c2(i):
                coarse[pl.ds(pl.multiple_of(i * _L, _L), _L)] = zeros16

        with jax.named_scope("pass2"):
            @plsc.parallel_loop(0, _NVEC, unroll=_UNROLL)
            def _pass2(i):
                off = pl.multiple_of(i * _L, _L)
                key = lax.bitcast_convert_type(
                    row_buf[pl.ds(off, _L)], jnp.int32
                )
                m = lax.shift_right_arithmetic(key, 18) == b1s
                idx = lax.shift_right_logical(key, 4) & jnp.int32(0x3FFF)
                plsc.addupdate_scatter(fine, [idx], ones, mask=m)
                plsc.addupdate_scatter(
                    coarse, [lax.shift_right_logical(idx, 4)], ones, mask=m
                )

        with jax.named_scope("scan2"):
            cc, above_c = _walk(coarse, jnp.int32(_CBINS // _L - 1), k2)
            fv = fine[pl.ds(pl.multiple_of(cc * _L, _L), _L)]
            cf, above2 = _suffix_find(fv, above_c, k2)
            b2 = cc * _L + cf
            k3 = k2 - above2
            p28 = lax.shift_left(b1s, 14) | b2

        fine[pl.ds(0, _L)] = zeros16

        with jax.named_scope("pass3"):
            @plsc.parallel_loop(0, _NVEC, unroll=_UNROLL)
            def _pass3(i):
                off = pl.multiple_of(i * _L, _L)
                key = lax.bitcast_convert_type(
                    row_buf[pl.ds(off, _L)], jnp.int32
                )
                m = lax.shift_right_arithmetic(key, 4) == p28
                plsc.addupdate_scatter(
                    fine, [key & jnp.int32(0xF)], ones, mask=m
                )

        with jax.named_scope("scan3"):
            b3, _ = _suffix_find(fine[pl.ds(0, _L)], jnp.int32(0), k3)
            key_final = lax.shift_left(p28, 4) | b3

        return jnp.where(lanes == j, key_final, res_keys)

    res_keys = lax.fori_loop(
        0, _ROWS_PER_TILE, row_body, jnp.zeros((_L,), jnp.int32)
    )

    s = res_keys ^ (lax.shift_right_arithmetic(res_keys, 31) & jnp.int32(0x7FFFFFFF))
    res_buf[...] = lax.bitcast_convert_type(s, jnp.float32)
    pltpu.sync_copy(res_buf, out_hbm.at[wid])


_sc_thresholds = functools.partial(
    pl.kernel,
    out_type=jax.ShapeDtypeStruct((_NW, _L), jnp.float32),
    mesh=plsc.VectorSubcoreMesh(core_axis_name="c", subcore_axis_name="s"),
    compiler_params=pltpu.CompilerParams(needs_layout_passes=False),
    scratch_types=[
        pltpu.VMEM((_N,), jnp.float32),
        pltpu.VMEM((_FBINS,), jnp.int32),
        pltpu.VMEM((_CBINS,), jnp.int32),
        pltpu.VMEM((_L,), jnp.float32),
    ],
)(_sc_body)


def _stats_body(nbf_ref, sim_ref, mn_ref, mx_ref, ms_ref, xs_ref):
    i = pl.program_id(0)
    nb = nbf_ref[...]
    sm = sim_ref[...]
    mn = jnp.broadcast_to(jnp.min(nb, axis=0, keepdims=True), (8, _B))
    mx = jnp.broadcast_to(jnp.max(nb, axis=0, keepdims=True), (8, _B))
    ms = jnp.broadcast_to(jnp.min(sm, axis=0, keepdims=True), (8, _B))
    xs = jnp.broadcast_to(jnp.max(sm, axis=0, keepdims=True), (8, _B))

    @pl.when(i == 0)
    def _():
        mn_ref[...] = mn
        mx_ref[...] = mx
        ms_ref[...] = ms
        xs_ref[...] = xs

    @pl.when(i > 0)
    def _():
        mn_ref[...] = jnp.minimum(mn_ref[...], mn)
        mx_ref[...] = jnp.maximum(mx_ref[...], mx)
        ms_ref[...] = jnp.minimum(ms_ref[...], ms)
        xs_ref[...] = jnp.maximum(xs_ref[...], xs)


def _thresh_body(nbf_ref, mn_ref, mx_ref, traw_ref, out_ref):
    i = pl.program_id(0)
    mn = mn_ref[0:1, :]
    den = mx_ref[0:1, :] - mn
    nb = nbf_ref[...]
    nbn = (nb - mn) / den
    cand = jnp.broadcast_to(
        jnp.max(
            jnp.where(nb <= traw_ref[0:1, :], nbn, -jnp.inf),
            axis=0,
            keepdims=True,
        ),
        (8, _B),
    )

    @pl.when(i == 0)
    def _():
        out_ref[...] = cand

    @pl.when(i > 0)
    def _():
        out_ref[...] = jnp.maximum(out_ref[...], cand)


def _out_body(nbf_ref, sim_ref, mn_ref, mx_ref, ms_ref, xs_ref, th_ref, out_ref):
    mn = mn_ref[0:1, :]
    den_n = mx_ref[0:1, :] - mn
    ms = ms_ref[0:1, :]
    den_s = xs_ref[0:1, :] - ms
    th = th_ref[0:1, :]
    nb = nbf_ref[...]
    sm = sim_ref[...]
    nbn = (nb - mn) / den_n
    smn = (sm - ms) / den_s
    out_ref[...] = nbn + jnp.where(nbn >= th, 1000.0 * (1.0 + smn), 0.0)


@jax.jit
def kernel(nbf_score, simkgc_score):
    b, n = nbf_score.shape
    nbf_t = nbf_score.T
    sim_t = simkgc_score.T

    t_tiles = _sc_thresholds(nbf_score)
    t_raw = jnp.tile(t_tiles[:, :_ROWS_PER_TILE].reshape(1, b), (8, 1))

    chunk_spec = pl.BlockSpec((_TCCHUNK, b), lambda i: (i, 0))
    small_spec = pl.BlockSpec((8, b), lambda i: (0, 0))
    s8 = jax.ShapeDtypeStruct((8, b), jnp.float32)

    mn, mx, ms, xs = pl.pallas_call(
        _stats_body,
        grid=(_TCGRID,),
        in_specs=[chunk_spec, chunk_spec],
        out_specs=[small_spec] * 4,
        out_shape=[s8] * 4,
    )(nbf_t, sim_t)

    thresh = pl.pallas_call(
        _thresh_body,
        grid=(_TCGRID,),
        in_specs=[chunk_spec] + [small_spec] * 3,
        out_specs=small_spec,
        out_shape=s8,
    )(nbf_t, mn, mx, t_raw)

    out_t = pl.pallas_call(
        _out_body,
        grid=(_TCGRID,),
        in_specs=[chunk_spec, chunk_spec] + [small_spec] * 5,
        out_specs=chunk_spec,
        out_shape=jax.ShapeDtypeStruct((n, b), jnp.float32),
    )(nbf_t, sim_t, mn, mx, ms, xs, thresh)

    return out_t.T


# speedup vs baseline: 1.0211x; 1.0211x over previous
import functools

import jax
import jax.numpy as jnp
from jax import lax
from jax.experimental import pallas as pl
from jax.experimental.pallas import tpu as pltpu
from jax.experimental.pallas import tpu_sc as plsc

_K = 100
_B = 128
_N = 100000

_NC, _NS, _L = 2, 16, 16
_NW = _NC * _NS
_ROWS_PER_TILE = _B // _NW
_FBINS = 1 << 14
_CBINS = _FBINS // _L
_UNROLL = 10
_NVEC = _N // _L

_TCCHUNK = 10000
_TCGRID = _N // _TCCHUNK


def _keys_of(v):
    s = lax.bitcast_convert_type(v, jnp.int32)
    return s ^ (lax.shift_right_arithmetic(s, 31) & jnp.int32(0x7FFFFFFF))


def _suffix_find(v, acc, k):
    suffix = lax.rev(jnp.cumsum(lax.rev(v, (0,))), (0,))
    ok = (acc + suffix) >= k
    c = jnp.sum(ok.astype(jnp.int32))
    above = acc + jnp.sum(jnp.where(ok, 0, v))
    return c - 1, above


def _walk(hist, start_chunk, k):
    def chunk_sum(j):
        off = pl.multiple_of(j * _L, _L)
        return jnp.sum(hist[pl.ds(off, _L)])

    def cond(st):
        j, acc, s = st
        return acc + s < k

    def body(st):
        j, acc, s = st
        return j - 1, acc + s, chunk_sum(j - 1)

    j, acc, _ = lax.while_loop(
        cond, body, (start_chunk, jnp.int32(0), chunk_sum(start_chunk))
    )
    v = hist[pl.ds(pl.multiple_of(j * _L, _L), _L)]
    ci, above = _suffix_find(v, acc, k)
    return j * _L + ci, above


def _sc_body(nbf_hbm, out_hbm, row_buf, fine, coarse, res_buf):
    wid = lax.axis_index("s") * _NC + lax.axis_index("c")
    lanes = lax.iota(jnp.int32, _L)
    ones = jnp.ones((_L,), jnp.int32)
    zeros16 = jnp.zeros((_L,), jnp.int32)

    def row_body(j, res_keys):
        row = wid * _ROWS_PER_TILE + j
        with jax.named_scope("dma_row"):
            pltpu.sync_copy(nbf_hbm.at[row], row_buf)

        with jax.named_scope("clear"):
            @plsc.parallel_loop(0, _FBINS // _L, unroll=8)
            def _clear_f(i):
                fine[pl.ds(pl.multiple_of(i * _L, _L), _L)] = zeros16

        with jax.named_scope("pass1"):
            @plsc.parallel_loop(
                0, _NVEC, unroll=_UNROLL, carry=jnp.zeros((_L,), jnp.int32)
            )
            def mx(i, mx):
                off = pl.multiple_of(i * _L, _L)
                key = _keys_of(row_buf[pl.ds(off, _L)])
                row_buf[pl.ds(off, _L)] = lax.bitcast_convert_type(
                    key, jnp.float32
                )
                idx = lax.shift_right_arithmetic(key, 18) + jnp.int32(8192)
                plsc.addupdate_scatter(fine, [idx], ones)
                return jnp.maximum(mx, idx)

        with jax.named_scope("scan1"):
            start = lax.shift_right_logical(jnp.max(mx), 4)
            b1, above1 = _walk(fine, start, jnp.int32(_K))
            k2 = jnp.int32(_K) - above1
            b1s = b1 - jnp.int32(8192)

        with jax.named_scope("clear2"):
            @plsc.parallel_loop(0, _FBINS // _L, unroll=8)
            def _clear_f2(i):
                fine[pl.ds(pl.multiple_of(i * _L, _L), _L)] = zeros16

            @plsc.parallel_loop(0, _CBINS // _L, unroll=8)
            def _clear_c2(i):
                coarse[pl.ds(pl.multiple_of(i * _L, _L), _L)] = zeros16

        with jax.named_scope("pass2"):
            @plsc.parallel_loop(0, _NVEC, unroll=_UNROLL)
            def _pass2(i):
                off = pl.multiple_of(i * _L, _L)
                key = lax.bitcast_convert_type(
                    row_buf[pl.ds(off, _L)], jnp.int32
                )
                m = lax.shift_right_arithmetic(key, 18) == b1s
                idx = lax.shift_right_logical(key, 4) & jnp.int32(0x3FFF)
                plsc.addupdate_scatter(fine, [idx], ones, mask=m)
                plsc.addupdate_scatter(
                    coarse, [lax.shift_right_logical(idx, 4)], ones, mask=m
                )

        with jax.named_scope("scan2"):
            cc, above_c = _walk(coarse, jnp.int32(_CBINS // _L - 1), k2)
            fv = fine[pl.ds(pl.multiple_of(cc * _L, _L), _L)]
            cf, above2 = _suffix_find(fv, above_c, k2)
            b2 = cc * _L + cf
            k3 = k2 - above2
            p28 = lax.shift_left(b1s, 14) | b2

        fine[pl.ds(0, _L)] = zeros16

        with jax.named_scope("pass3"):
            @plsc.parallel_loop(0, _NVEC, unroll=_UNROLL)
            def _pass3(i):
                off = pl.multiple_of(i * _L, _L)
                key = lax.bitcast_convert_type(
                    row_buf[pl.ds(off, _L)], jnp.int32
                )
                m = lax.shift_right_arithmetic(key, 4) == p28
                plsc.addupdate_scatter(
                    fine, [key & jnp.int32(0xF)], ones, mask=m
                )

        with jax.named_scope("scan3"):
            b3, _ = _suffix_find(fine[pl.ds(0, _L)], jnp.int32(0), k3)
            key_final = lax.shift_left(p28, 4) | b3

        return jnp.where(lanes == j, key_final, res_keys)

    res_keys = lax.fori_loop(
        0, _ROWS_PER_TILE, row_body, jnp.zeros((_L,), jnp.int32)
    )

    s = res_keys ^ (lax.shift_right_arithmetic(res_keys, 31) & jnp.int32(0x7FFFFFFF))
    res_buf[...] = lax.bitcast_convert_type(s, jnp.float32)
    pltpu.sync_copy(res_buf, out_hbm.at[wid])


_sc_thresholds = functools.partial(
    pl.kernel,
    out_type=jax.ShapeDtypeStruct((_NW, _L), jnp.float32),
    mesh=plsc.VectorSubcoreMesh(core_axis_name="c", subcore_axis_name="s"),
    compiler_params=pltpu.CompilerParams(needs_layout_passes=False),
    scratch_types=[
        pltpu.VMEM((_N,), jnp.float32),
        pltpu.VMEM((_FBINS,), jnp.int32),
        pltpu.VMEM((_CBINS,), jnp.int32),
        pltpu.VMEM((_L,), jnp.float32),
    ],
)(_sc_body)


def _stats_body(nbf_ref, sim_ref, mn_ref, mx_ref, ms_ref, xs_ref):
    i = pl.program_id(0)
    nb = nbf_ref[...]
    sm = sim_ref[...]
    mn = jnp.broadcast_to(jnp.min(nb, axis=0, keepdims=True), (8, _B))
    mx = jnp.broadcast_to(jnp.max(nb, axis=0, keepdims=True), (8, _B))
    ms = jnp.broadcast_to(jnp.min(sm, axis=0, keepdims=True), (8, _B))
    xs = jnp.broadcast_to(jnp.max(sm, axis=0, keepdims=True), (8, _B))

    @pl.when(i == 0)
    def _():
        mn_ref[...] = mn
        mx_ref[...] = mx
        ms_ref[...] = ms
        xs_ref[...] = xs

    @pl.when(i > 0)
    def _():
        mn_ref[...] = jnp.minimum(mn_ref[...], mn)
        mx_ref[...] = jnp.maximum(mx_ref[...], mx)
        ms_ref[...] = jnp.minimum(ms_ref[...], ms)
        xs_ref[...] = jnp.maximum(xs_ref[...], xs)


def _thresh_body(nbf_ref, mn_ref, mx_ref, traw_ref, out_ref):
    i = pl.program_id(0)
    mn = mn_ref[0:1, :]
    den = mx_ref[0:1, :] - mn
    nb = nbf_ref[...]
    nbn = (nb - mn) / den
    cand = jnp.broadcast_to(
        jnp.max(
            jnp.where(nb <= traw_ref[0:1, :], nbn, -jnp.inf),
            axis=0,
            keepdims=True,
        ),
        (8, _B),
    )

    @pl.when(i == 0)
    def _():
        out_ref[...] = cand

    @pl.when(i > 0)
    def _():
        out_ref[...] = jnp.maximum(out_ref[...], cand)


def _out_body(nbf_ref, sim_ref, mn_ref, mx_ref, ms_ref, xs_ref, th_ref, out_ref):
    mn = mn_ref[0:1, :]
    den_n = mx_ref[0:1, :] - mn
    ms = ms_ref[0:1, :]
    den_s = xs_ref[0:1, :] - ms
    th = th_ref[0:1, :]
    nb = nbf_ref[...]
    sm = sim_ref[...]
    nbn = (nb - mn) / den_n
    smn = (sm - ms) / den_s
    out_ref[...] = nbn + jnp.where(nbn >= th, 1000.0 * (1.0 + smn), 0.0)


@jax.jit
def kernel(nbf_score, simkgc_score):
    b, n = nbf_score.shape
    nbf_t = nbf_score.T
    sim_t = simkgc_score.T

    t_tiles = _sc_thresholds(nbf_score)
    t_raw = jnp.tile(t_tiles[:, :_ROWS_PER_TILE].reshape(1, b), (8, 1))

    chunk_spec = pl.BlockSpec((_TCCHUNK, b), lambda i: (i, 0))
    small_spec = pl.BlockSpec((8, b), lambda i: (0, 0))
    s8 = jax.ShapeDtypeStruct((8, b), jnp.float32)

    mn, mx, ms, xs = pl.pallas_call(
        _stats_body,
        grid=(_TCGRID,),
        in_specs=[chunk_spec, chunk_spec],
        out_specs=[small_spec] * 4,
        out_shape=[s8] * 4,
    )(nbf_t, sim_t)

    thresh = pl.pallas_call(
        _thresh_body,
        grid=(_TCGRID,),
        in_specs=[chunk_spec] + [small_spec] * 3,
        out_specs=small_spec,
        out_shape=s8,
    )(nbf_t, mn, mx, t_raw)

    out_t = pl.pallas_call(
        _out_body,
        grid=(_TCGRID,),
        in_specs=[chunk_spec, chunk_spec] + [small_spec] * 5,
        out_specs=chunk_spec,
        out_shape=jax.ShapeDtypeStruct((n, b), jnp.float32),
    )(nbf_t, sim_t, mn, mx, ms, xs, thresh)

    return out_t.T


# speedup vs baseline: 1.0211x; 1.0000x over previous
import functools

import jax
import jax.numpy as jnp
from jax import lax
from jax.experimental import pallas as pl
from jax.experimental.pallas import tpu as pltpu
from jax.experimental.pallas import tpu_sc as plsc

_K = 100
_B = 128
_N = 100000

_NC, _NS, _L = 2, 16, 16
_NW = _NC * _NS
_ROWS_PER_TILE = _B // _NW
_FBINS = 1 << 14
_CBINS = _FBINS // _L
_UNROLL = 10
_NVEC = _N // _L

_TCCHUNK = 10000
_TCGRID = _N // _TCCHUNK


def _keys_of(v):
    s = lax.bitcast_convert_type(v, jnp.int32)
    return s ^ (lax.shift_right_arithmetic(s, 31) & jnp.int32(0x7FFFFFFF))


def _suffix_find(v, acc, k):
    suffix = lax.rev(jnp.cumsum(lax.rev(v, (0,))), (0,))
    ok = (acc + suffix) >= k
    c = jnp.sum(ok.astype(jnp.int32))
    above = acc + jnp.sum(jnp.where(ok, 0, v))
    return c - 1, above


def _walk(hist, start_chunk, k):
    def chunk_sum(j):
        off = pl.multiple_of(j * _L, _L)
        return jnp.sum(hist[pl.ds(off, _L)])

    def cond(st):
        j, acc, s = st
        return acc + s < k

    def body(st):
        j, acc, s = st
        return j - 1, acc + s, chunk_sum(j - 1)

    j, acc, _ = lax.while_loop(
        cond, body, (start_chunk, jnp.int32(0), chunk_sum(start_chunk))
    )
    v = hist[pl.ds(pl.multiple_of(j * _L, _L), _L)]
    ci, above = _suffix_find(v, acc, k)
    return j * _L + ci, above


def _sc_body(nbf_hbm, out_hbm, row_buf, fine, coarse, res_buf):
    wid = lax.axis_index("s") * _NC + lax.axis_index("c")
    lanes = lax.iota(jnp.int32, _L)
    ones = jnp.ones((_L,), jnp.int32)
    zeros16 = jnp.zeros((_L,), jnp.int32)

    def row_body(j, res_keys):
        row = wid * _ROWS_PER_TILE + j
        with jax.named_scope("dma_row"):
            pltpu.sync_copy(nbf_hbm.at[row], row_buf)

        with jax.named_scope("clear"):
            @plsc.parallel_loop(0, _FBINS // _L, unroll=8)
            def _clear_f(i):
                fine[pl.ds(pl.multiple_of(i * _L, _L), _L)] = zeros16

        with jax.named_scope("pass1"):
            @plsc.parallel_loop(
                0, _NVEC, unroll=_UNROLL, carry=jnp.zeros((_L,), jnp.int32)
            )
            def mx(i, mx):
                off = pl.multiple_of(i * _L, _L)
                key = _keys_of(row_buf[pl.ds(off, _L)])
                row_buf[pl.ds(off, _L)] = lax.bitcast_convert_type(
                    key, jnp.float32
                )
                idx = lax.shift_right_arithmetic(key, 18) + jnp.int32(8192)
                plsc.addupdate_scatter(fine, [idx], ones)
                return jnp.maximum(mx, idx)

        with jax.named_scope("scan1"):
            start = lax.shift_right_logical(jnp.max(mx), 4)
            b1, above1 = _walk(fine, start, jnp.int32(_K))
            k2 = jnp.int32(_K) - above1
            b1s = b1 - jnp.int32(8192)

        with jax.named_scope("clear2"):
            @plsc.parallel_loop(0, _FBINS // _L, unroll=8)
            def _clear_f2(i):
                fine[pl.ds(pl.multiple_of(i * _L, _L), _L)] = zeros16

            @plsc.parallel_loop(0, _CBINS // _L, unroll=8)
            def _clear_c2(i):
                coarse[pl.ds(pl.multiple_of(i * _L, _L), _L)] = zeros16

        with jax.named_scope("pass2"):
            @plsc.parallel_loop(0, _NVEC, unroll=_UNROLL)
            def _pass2(i):
                off = pl.multiple_of(i * _L, _L)
                key = lax.bitcast_convert_type(
                    row_buf[pl.ds(off, _L)], jnp.int32
                )
                m = lax.shift_right_arithmetic(key, 18) == b1s
                idx = lax.shift_right_logical(key, 4) & jnp.int32(0x3FFF)
                plsc.addupdate_scatter(fine, [idx], ones, mask=m)
                plsc.addupdate_scatter(
                    coarse, [lax.shift_right_logical(idx, 4)], ones, mask=m
                )

        with jax.named_scope("scan2"):
            cc, above_c = _walk(coarse, jnp.int32(_CBINS // _L - 1), k2)
            fv = fine[pl.ds(pl.multiple_of(cc * _L, _L), _L)]
            cf, above2 = _suffix_find(fv, above_c, k2)
            b2 = cc * _L + cf
            k3 = k2 - above2
            p28 = lax.shift_left(b1s, 14) | b2

        fine[pl.ds(0, _L)] = zeros16

        with jax.named_scope("pass3"):
            @plsc.parallel_loop(0, _NVEC, unroll=_UNROLL)
            def _pass3(i):
                off = pl.multiple_of(i * _L, _L)
                key = lax.bitcast_convert_type(
                    row_buf[pl.ds(off, _L)], jnp.int32
                )
                m = lax.shift_right_arithmetic(key, 4) == p28
                plsc.addupdate_scatter(
                    fine, [key & jnp.int32(0xF)], ones, mask=m
                )

        with jax.named_scope("scan3"):
            b3, _ = _suffix_find(fine[pl.ds(0, _L)], jnp.int32(0), k3)
            key_final = lax.shift_left(p28, 4) | b3

        return jnp.where(lanes == j, key_final, res_keys)

    res_keys = lax.fori_loop(
        0, _ROWS_PER_TILE, row_body, jnp.zeros((_L,), jnp.int32)
    )

    s = res_keys ^ (lax.shift_right_arithmetic(res_keys, 31) & jnp.int32(0x7FFFFFFF))
    res_buf[...] = lax.bitcast_convert_type(s, jnp.float32)
    pltpu.sync_copy(res_buf, out_hbm.at[wid])


@functools.cache
def _sc_thresholds_fn():
    return functools.partial(
        pl.kernel,
        out_type=jax.ShapeDtypeStruct((_NW, _L), jnp.float32),
        mesh=plsc.VectorSubcoreMesh(core_axis_name="c", subcore_axis_name="s"),
        compiler_params=pltpu.CompilerParams(needs_layout_passes=False),
        scratch_types=[
            pltpu.VMEM((_N,), jnp.float32),
            pltpu.VMEM((_FBINS,), jnp.int32),
            pltpu.VMEM((_CBINS,), jnp.int32),
            pltpu.VMEM((_L,), jnp.float32),
        ],
    )(_sc_body)


def _stats_body(nbf_ref, sim_ref, mn_ref, mx_ref, ms_ref, xs_ref):
    i = pl.program_id(0)
    nb = nbf_ref[...]
    sm = sim_ref[...]
    mn = jnp.broadcast_to(jnp.min(nb, axis=0, keepdims=True), (8, _B))
    mx = jnp.broadcast_to(jnp.max(nb, axis=0, keepdims=True), (8, _B))
    ms = jnp.broadcast_to(jnp.min(sm, axis=0, keepdims=True), (8, _B))
    xs = jnp.broadcast_to(jnp.max(sm, axis=0, keepdims=True), (8, _B))

    @pl.when(i == 0)
    def _():
        mn_ref[...] = mn
        mx_ref[...] = mx
        ms_ref[...] = ms
        xs_ref[...] = xs

    @pl.when(i > 0)
    def _():
        mn_ref[...] = jnp.minimum(mn_ref[...], mn)
        mx_ref[...] = jnp.maximum(mx_ref[...], mx)
        ms_ref[...] = jnp.minimum(ms_ref[...], ms)
        xs_ref[...] = jnp.maximum(xs_ref[...], xs)


def _thresh_body(nbf_ref, mn_ref, mx_ref, traw_ref, out_ref):
    i = pl.program_id(0)
    mn = mn_ref[0:1, :]
    den = mx_ref[0:1, :] - mn
    nb = nbf_ref[...]
    nbn = (nb - mn) / den
    cand = jnp.broadcast_to(
        jnp.max(
            jnp.where(nb <= traw_ref[0:1, :], nbn, -jnp.inf),
            axis=0,
            keepdims=True,
        ),
        (8, _B),
    )

    @pl.when(i == 0)
    def _():
        out_ref[...] = cand

    @pl.when(i > 0)
    def _():
        out_ref[...] = jnp.maximum(out_ref[...], cand)


def _out_body(nbf_ref, sim_ref, mn_ref, mx_ref, ms_ref, xs_ref, th_ref, out_ref):
    mn = mn_ref[0:1, :]
    den_n = mx_ref[0:1, :] - mn
    ms = ms_ref[0:1, :]
    den_s = xs_ref[0:1, :] - ms
    th = th_ref[0:1, :]
    nb = nbf_ref[...]
    sm = sim_ref[...]
    nbn = (nb - mn) / den_n
    smn = (sm - ms) / den_s
    out_ref[...] = nbn + jnp.where(nbn >= th, 1000.0 * (1.0 + smn), 0.0)


@jax.jit
def kernel(nbf_score, simkgc_score):
    b, n = nbf_score.shape
    nbf_t = nbf_score.T
    sim_t = simkgc_score.T

    t_tiles = _sc_thresholds_fn()(nbf_score)
    t_raw = jnp.tile(t_tiles[:, :_ROWS_PER_TILE].reshape(1, b), (8, 1))

    chunk_spec = pl.BlockSpec((_TCCHUNK, b), lambda i: (i, 0))
    small_spec = pl.BlockSpec((8, b), lambda i: (0, 0))
    s8 = jax.ShapeDtypeStruct((8, b), jnp.float32)

    mn, mx, ms, xs = pl.pallas_call(
        _stats_body,
        grid=(_TCGRID,),
        in_specs=[chunk_spec, chunk_spec],
        out_specs=[small_spec] * 4,
        out_shape=[s8] * 4,
    )(nbf_t, sim_t)

    thresh = pl.pallas_call(
        _thresh_body,
        grid=(_TCGRID,),
        in_specs=[chunk_spec] + [small_spec] * 3,
        out_specs=small_spec,
        out_shape=s8,
    )(nbf_t, mn, mx, t_raw)

    out_t = pl.pallas_call(
        _out_body,
        grid=(_TCGRID,),
        in_specs=[chunk_spec, chunk_spec] + [small_spec] * 5,
        out_specs=chunk_spec,
        out_shape=jax.ShapeDtypeStruct((n, b), jnp.float32),
    )(nbf_t, sim_t, mn, mx, ms, xs, thresh)

    return out_t.T
